# Initial kernel scaffold; baseline (speedup 1.0000x reference)
import jax
import jax.numpy as jnp
from jax.experimental import pallas as pl


def kernel(v_j, u_k, emd_table, h_table):
    raise NotImplementedError("write your pallas kernel here")


# speedup vs baseline: 1.7939x; 1.7939x over previous
import jax
import jax.numpy as jnp
from jax.experimental import pallas as pl
from jax.experimental.pallas import tpu as pltpu

jax.config.update("jax_enable_x64", True)

_EMD_DIM = 32
_NUM_V = 1000000
_MAX_STEPS = 22


def _walk_body(su_ref, emd_hbm, h_hbm, out_ref, emd_v, rows_v, emd_sem, row_sem):
    v = su_ref[0]
    u = su_ref[1]
    emd_cp = pltpu.make_async_copy(emd_hbm.at[pl.ds(v, 1)], emd_v, emd_sem)
    emd_cp.start()

    t = 2 * (_NUM_V - 1 + u)
    n = jnp.int32(0)
    cps = []
    for k in range(_MAX_STEPS):
        active = (t != 0).astype(jnp.int32)
        n = n + active
        t_raw = jnp.where(t % 4 == 0, t // 2 - 1, t // 2)
        t = jnp.where(t != 0, t_raw, 0)
        i = t // 2
        cp = pltpu.make_async_copy(
            h_hbm.at[pl.ds(i, 1)], rows_v.at[pl.ds(k, 1)], row_sem
        )
        cp.start()
        cps.append(cp)

    emd_cp.wait()
    for cp in cps:
        cp.wait()

    emd32 = emd_v[...].astype(jnp.float32)
    rows32 = rows_v[...].astype(jnp.float32)
    dots = jnp.sum(rows32 * emd32, axis=1, keepdims=True)
    ls = jnp.minimum(dots, 0.0) - jnp.log1p(jnp.exp(-jnp.abs(dots)))
    step_ids = jax.lax.broadcasted_iota(jnp.int32, (_MAX_STEPS, 1), 0)
    factors = jnp.where(step_ids < n, ls, jnp.float32(1.0))
    p = jnp.concatenate(
        [factors, jnp.ones((32 - _MAX_STEPS, 1), jnp.float32)], axis=0
    )
    for half in (16, 8, 4, 2, 1):
        p = p[:half] * p[half : 2 * half]
    out_ref[...] = -p


def kernel(v_j, u_k, emd_table, h_table):
    emd_table = emd_table.astype(jnp.float32)
    h_table = h_table.astype(jnp.float32)
    su = jnp.stack([v_j, u_k]).astype(jnp.int32)
    out = pl.pallas_call(
        _walk_body,
        in_specs=[
            pl.BlockSpec(memory_space=pltpu.SMEM),
            pl.BlockSpec(memory_space=pltpu.HBM),
            pl.BlockSpec(memory_space=pltpu.HBM),
        ],
        out_specs=pl.BlockSpec(memory_space=pltpu.VMEM),
        out_shape=jax.ShapeDtypeStruct((1, 1), jnp.float32),
        scratch_shapes=[
            pltpu.VMEM((1, _EMD_DIM), emd_table.dtype),
            pltpu.VMEM((_MAX_STEPS, _EMD_DIM), h_table.dtype),
            pltpu.SemaphoreType.DMA,
            pltpu.SemaphoreType.DMA,
        ],
    )(su, emd_table, h_table)
    return out[0, 0].astype(jnp.float64)


# speedup vs baseline: 4.9144x; 2.7394x over previous
import jax
import jax.numpy as jnp
from jax.experimental import pallas as pl
from jax.experimental.pallas import tpu as pltpu

jax.config.update("jax_enable_x64", True)

_EMD_DIM = 32
_NUM_V = 1000000
_MAX_STEPS = 22
_WIN = 128
_N_WROWS = _NUM_V * _EMD_DIM // _WIN


def _walk_body(su_ref, emd_hbm, h_hbm, out_ref, emd_v, rows_v, emd_sem, row_sem):
    v = su_ref[0]
    u = su_ref[1]
    e_wrow = jnp.minimum(v >> 2, _N_WROWS - 1)
    o_e = _EMD_DIM * v - _WIN * e_wrow
    emd_cp = pltpu.make_async_copy(
        emd_hbm.at[pl.ds(e_wrow, 1)], emd_v, emd_sem
    )
    emd_cp.start()

    iota = jax.lax.broadcasted_iota(jnp.int32, (_MAX_STEPS, 1), 0)
    t = 2 * (_NUM_V - 1 + u)
    n = jnp.int32(0)
    o_vec = jnp.zeros((_MAX_STEPS, 1), jnp.int32)
    cps = []
    for k in range(_MAX_STEPS):
        active = (t != 0).astype(jnp.int32)
        n = n + active
        t_raw = jnp.where(t % 4 == 0, t // 2 - 1, t // 2)
        t = jnp.where(t != 0, t_raw, 0)
        i = t // 2
        wrow = jnp.minimum(i >> 2, _N_WROWS - 1)
        o_vec = o_vec + jnp.where(iota == k, _EMD_DIM * i - _WIN * wrow, 0)
        cp = pltpu.make_async_copy(
            h_hbm.at[pl.ds(wrow, 1)], rows_v.at[pl.ds(k, 1)], row_sem
        )
        cp.start()
        cps.append(cp)

    emd_cp.wait()
    for cp in cps:
        cp.wait()

    ew = emd_v[...]
    e_sel = jnp.where(
        o_e == 0,
        ew[:, 0:32],
        jnp.where(
            o_e == 32,
            ew[:, 32:64],
            jnp.where(o_e == 64, ew[:, 64:96], ew[:, 96:128]),
        ),
    )
    rows = rows_v[...]
    d0 = jnp.sum(rows[:, 0:32] * e_sel, axis=1, keepdims=True)
    d1 = jnp.sum(rows[:, 32:64] * e_sel, axis=1, keepdims=True)
    d2 = jnp.sum(rows[:, 64:96] * e_sel, axis=1, keepdims=True)
    d3 = jnp.sum(rows[:, 96:128] * e_sel, axis=1, keepdims=True)
    dots = jnp.where(
        o_vec == 0, d0, jnp.where(o_vec == 32, d1, jnp.where(o_vec == 64, d2, d3))
    )
    ls = jnp.minimum(dots, 0.0) - jnp.log1p(jnp.exp(-jnp.abs(dots)))
    factors = jnp.where(iota < n, ls, jnp.float32(1.0))
    p = jnp.concatenate(
        [factors, jnp.ones((32 - _MAX_STEPS, 1), jnp.float32)], axis=0
    )
    for half in (16, 8, 4, 2, 1):
        p = p[:half] * p[half : 2 * half]
    out_ref[...] = -p


def kernel(v_j, u_k, emd_table, h_table):
    e2 = emd_table.astype(jnp.float32).reshape(_N_WROWS, _WIN)
    h_flat = h_table.astype(jnp.float32).reshape(-1)
    h2 = jnp.concatenate([h_flat, jnp.zeros(_EMD_DIM, jnp.float32)]).reshape(
        _N_WROWS, _WIN
    )
    su = jnp.stack([v_j, u_k]).astype(jnp.int32)
    out = pl.pallas_call(
        _walk_body,
        in_specs=[
            pl.BlockSpec(memory_space=pltpu.SMEM),
            pl.BlockSpec(memory_space=pltpu.HBM),
            pl.BlockSpec(memory_space=pltpu.HBM),
        ],
        out_specs=pl.BlockSpec(memory_space=pltpu.VMEM),
        out_shape=jax.ShapeDtypeStruct((1, 1), jnp.float32),
        scratch_shapes=[
            pltpu.VMEM((1, _WIN), jnp.float32),
            pltpu.VMEM((_MAX_STEPS, _WIN), jnp.float32),
            pltpu.SemaphoreType.DMA,
            pltpu.SemaphoreType.DMA,
        ],
    )(su, e2, h2)
    return out[0, 0].astype(jnp.float64)
